# Initial kernel scaffold; baseline (speedup 1.0000x reference)
import jax
import jax.numpy as jnp
from jax.experimental import pallas as pl


def kernel(cls_embed, scores, fc1_w, fc1_b, fc2_w, fc2_b, w_gate, exp_w1, exp_b1, exp_w2, exp_b2, tower_w1, tower_b1, tower_w2, tower_b2):
    raise NotImplementedError("write your pallas kernel here")


# speedup vs baseline: 2.3257x; 2.3257x over previous
import functools

import jax
import jax.numpy as jnp
from jax import lax
from jax.experimental import pallas as pl

N_TASKS = 3
NUM_EXPERTS = 8
TOP_K = 2
BZS = 256
N_CAND = 4
B = N_CAND * BZS


def _bottom_gate_kernel(x_ref, fc1_ref, b1_ref, fc2_ref, b2_ref, wg_ref,
                        h_ref, gates_ref, aux_ref):
    a0 = jnp.dot(x_ref[...], fc1_ref[...], preferred_element_type=jnp.float32)
    a0 = jnp.maximum(a0 + b1_ref[...], 0.0).astype(jnp.bfloat16)
    h = jnp.dot(a0, fc2_ref[...], preferred_element_type=jnp.float32)
    h = h + b2_ref[...]
    hb = h.astype(jnp.bfloat16)
    h_ref[...] = hb
    gl_all = jnp.dot(hb, wg_ref[...], preferred_element_type=jnp.float32)
    iota = lax.broadcasted_iota(jnp.int32, (B, NUM_EXPERTS), 1)
    aux = jnp.float32(0.0)
    for j in range(N_TASKS):
        gl = gl_all[:, j * NUM_EXPERTS:(j + 1) * NUM_EXPERTS]
        m1 = jnp.max(gl, axis=1, keepdims=True)
        idx1 = jnp.min(jnp.where(gl == m1, iota, NUM_EXPERTS), axis=1,
                       keepdims=True)
        masked = jnp.where(iota == idx1, -jnp.inf, gl)
        m2 = jnp.max(masked, axis=1, keepdims=True)
        idx2 = jnp.min(jnp.where(masked == m2, iota, NUM_EXPERTS), axis=1,
                       keepdims=True)
        g1 = 1.0 / (1.0 + jnp.exp(m2 - m1))
        g2 = 1.0 - g1
        gates_j = (jnp.where(iota == idx1, g1, 0.0)
                   + jnp.where(iota == idx2, g2, 0.0))
        gates_ref[j] = gates_j
        for c in range(N_CAND):
            imp = jnp.sum(gates_j[c * BZS:(c + 1) * BZS], axis=0)
            mean = jnp.mean(imp)
            var = jnp.mean((imp - mean) ** 2)
            aux = aux + 0.01 * var / (mean * mean + 1e-10)
    aux_ref[...] = jnp.reshape(aux, (1, 1))


def _expert_l1_kernel(h_ref, w1_ref, b1_ref, a1_ref):
    a0 = jnp.dot(h_ref[...], w1_ref[...], preferred_element_type=jnp.float32)
    a1_ref[...] = jnp.maximum(a0 + b1_ref[...], 0.0).astype(jnp.bfloat16)


def _expert_l2_kernel(a1_ref, w2_ref, b2_ref, g_ref, out_ref):
    e = pl.program_id(1)
    o = jnp.dot(a1_ref[...], w2_ref[...], preferred_element_type=jnp.float32)
    o = o + b2_ref[...]

    @pl.when(e == 0)
    def _():
        for j in range(N_TASKS):
            out_ref[j] = g_ref[:, j:j + 1] * o

    @pl.when(e != 0)
    def _():
        for j in range(N_TASKS):
            out_ref[j] += g_ref[:, j:j + 1] * o


def _tower_loss_kernel(moe_ref, tw1_ref, tb1_ref, tw2_ref, tb2_ref,
                       scores_ref, aux_ref, preds_ref, loss_ref):
    total = jnp.float32(0.0)
    for j in range(N_TASKS):
        t1 = jnp.dot(moe_ref[j], tw1_ref[j],
                     preferred_element_type=jnp.float32)
        t1 = jnp.maximum(t1 + tb1_ref[j:j + 1, :], 0.0)
        logits = jnp.sum(t1 * tw2_ref[j:j + 1, :], axis=1, keepdims=True)
        logits = logits + tb2_ref[j:j + 1, :]
        preds_ref[j] = 1.0 / (1.0 + jnp.exp(-logits))
        for i in range(N_CAND):
            s = scores_ref[:, i, j:j + 1]
            labels = (s == jnp.max(s)).astype(jnp.float32)
            lg = logits[i * BZS:(i + 1) * BZS]
            bce = jnp.mean(jnp.maximum(lg, 0.0) - lg * labels
                           + jnp.log1p(jnp.exp(-jnp.abs(lg))))
            total = total + bce
    loss_ref[...] = (aux_ref[...] + total) / (N_CAND * N_TASKS)


@functools.partial(jax.jit, static_argnums=())
def kernel(cls_embed, scores, fc1_w, fc1_b, fc2_w, fc2_b, w_gate,
           exp_w1, exp_b1, exp_w2, exp_b2, tower_w1, tower_b1, tower_w2,
           tower_b2):
    f32 = jnp.float32
    bf16 = jnp.bfloat16
    hidden = fc1_w.shape[1]
    ehidden = exp_w1.shape[2]
    thidden = tower_w1.shape[2]

    x = cls_embed.transpose(1, 0, 2).reshape(B, -1).astype(bf16)
    wg2 = jnp.transpose(w_gate, (1, 0, 2)).reshape(hidden,
                                                   N_TASKS * NUM_EXPERTS)

    h, gates, aux = pl.pallas_call(
        _bottom_gate_kernel,
        out_shape=(
            jax.ShapeDtypeStruct((B, hidden), bf16),
            jax.ShapeDtypeStruct((N_TASKS, B, NUM_EXPERTS), f32),
            jax.ShapeDtypeStruct((1, 1), f32),
        ),
    )(x, fc1_w.astype(bf16), fc1_b.reshape(1, -1), fc2_w.astype(bf16),
      fc2_b.reshape(1, -1), wg2.astype(bf16))

    a1 = pl.pallas_call(
        _expert_l1_kernel,
        grid=(NUM_EXPERTS,),
        in_specs=[
            pl.BlockSpec((B, hidden), lambda e: (0, 0)),
            pl.BlockSpec((None, hidden, ehidden), lambda e: (e, 0, 0)),
            pl.BlockSpec((None, 1, ehidden), lambda e: (e, 0, 0)),
        ],
        out_specs=pl.BlockSpec((None, B, ehidden), lambda e: (e, 0, 0)),
        out_shape=jax.ShapeDtypeStruct((NUM_EXPERTS, B, ehidden), bf16),
    )(h, exp_w1.astype(bf16), exp_b1.reshape(NUM_EXPERTS, 1, ehidden))

    g_t = jnp.transpose(gates, (2, 1, 0))

    n_col = hidden // 2
    moe = pl.pallas_call(
        _expert_l2_kernel,
        grid=(2, NUM_EXPERTS),
        in_specs=[
            pl.BlockSpec((None, B, ehidden), lambda n, e: (e, 0, 0)),
            pl.BlockSpec((None, ehidden, n_col), lambda n, e: (e, 0, n)),
            pl.BlockSpec((None, 1, n_col), lambda n, e: (e, 0, n)),
            pl.BlockSpec((None, B, N_TASKS), lambda n, e: (e, 0, 0)),
        ],
        out_specs=pl.BlockSpec((N_TASKS, B, n_col), lambda n, e: (0, 0, n)),
        out_shape=jax.ShapeDtypeStruct((N_TASKS, B, hidden), f32),
    )(a1, exp_w2.astype(bf16), exp_b2.reshape(NUM_EXPERTS, 1, hidden), g_t)

    scores_t = jnp.transpose(scores, (2, 0, 1))

    preds, loss = pl.pallas_call(
        _tower_loss_kernel,
        out_shape=(
            jax.ShapeDtypeStruct((N_TASKS, B, 1), f32),
            jax.ShapeDtypeStruct((1, 1), f32),
        ),
    )(moe.astype(bf16), tower_w1.astype(bf16), tower_b1,
      tower_w2.reshape(N_TASKS, thidden), tower_b2, scores_t, aux)

    preds_out = preds.reshape(N_TASKS, N_CAND, BZS).transpose(1, 0, 2)
    return loss.reshape(()), preds_out
